# baseline (speedup 1.0000x reference)
import functools

import jax
import jax.numpy as jnp
from jax import lax
from jax.experimental import pallas as pl
from jax.experimental.pallas import tpu as pltpu
from jax.experimental.pallas import tpu_sc as plsc

VOCAB = 8192
HIDDEN = 256
GEN_HIDDEN = 4096
NUM_TOKENS = 8192
HID_F32 = HIDDEN // 2

NUM_WORKERS = 32
TOK_PER_W = NUM_TOKENS // NUM_WORKERS
IDX_CHUNK = 128

TILE_M = 1024
GRID_M = NUM_TOKENS // TILE_M


def _sc_gather_body(ids_hbm, table_hbm, out_hbm, idx_v, rows_v, sem):
    wid = lax.axis_index("s") * 2 + lax.axis_index("c")
    base = wid * TOK_PER_W
    pltpu.sync_copy(ids_hbm.at[pl.ds(base, TOK_PER_W)], idx_v)
    for i in range(TOK_PER_W // 16):
        sl = pl.ds(i * 16, 16)
        idx_v[sl] = idx_v[sl] - 4
    copies = []
    for j in range(TOK_PER_W // IDX_CHUNK):
        sl = pl.ds(j * IDX_CHUNK, IDX_CHUNK)
        copies.append(
            pltpu.async_copy(table_hbm.at[idx_v.at[sl]], rows_v.at[sl], sem)
        )
    for c in copies:
        c.wait()
    pltpu.sync_copy(rows_v, out_hbm.at[pl.ds(base, TOK_PER_W)])


def _sc_gather(ids_flat, table_f32):
    mesh = plsc.VectorSubcoreMesh(core_axis_name="c", subcore_axis_name="s")
    fn = pl.kernel(
        _sc_gather_body,
        out_type=jax.ShapeDtypeStruct((NUM_TOKENS, HID_F32), jnp.float32),
        mesh=mesh,
        scratch_types=[
            pltpu.VMEM((TOK_PER_W,), jnp.int32),
            pltpu.VMEM((TOK_PER_W, HID_F32), jnp.float32),
            pltpu.SemaphoreType.DMA,
        ],
    )
    return fn(ids_flat, table_f32)


def _tc_mse_body(x_ref, w_ref, b_ref, l_ref, out_ref):
    i = pl.program_id(0)
    f32 = jnp.float32
    acc = lax.dot_general(
        x_ref[...], w_ref[...],
        dimension_numbers=(((1,), (1,)), ((), ())),
        preferred_element_type=f32,
    ).astype(jnp.bfloat16)
    feat = (acc.astype(f32) + b_ref[...].astype(f32)).astype(jnp.bfloat16)
    diff = feat - l_ref[...]
    dsq = diff * diff
    part = jnp.sum(dsq.astype(f32))

    @pl.when(i == 0)
    def _():
        out_ref[0, 0] = 0.0

    out_ref[0, 0] += part

    @pl.when(i == GRID_M - 1)
    def _():
        out_ref[0, 0] = out_ref[0, 0] * (1.0 / (NUM_TOKENS * HIDDEN))


def _tc_mse(x, w, b2d, labels):
    return pl.pallas_call(
        _tc_mse_body,
        grid=(GRID_M,),
        in_specs=[
            pl.BlockSpec((TILE_M, GEN_HIDDEN), lambda i: (i, 0)),
            pl.BlockSpec((HIDDEN, GEN_HIDDEN), lambda i: (0, 0)),
            pl.BlockSpec((1, HIDDEN), lambda i: (0, 0)),
            pl.BlockSpec((TILE_M, HIDDEN), lambda i: (i, 0)),
        ],
        out_specs=pl.BlockSpec(memory_space=pltpu.SMEM),
        out_shape=jax.ShapeDtypeStruct((1, 1), jnp.float32),
    )(x, w, b2d, labels)


def kernel(input_ids, generated_hidden_states, embedding, W, b):
    ids_flat = input_ids.reshape(NUM_TOKENS).astype(jnp.int32)
    table_f32 = lax.bitcast_convert_type(
        embedding.reshape(VOCAB, HID_F32, 2), jnp.float32
    )
    labels_f32 = _sc_gather(ids_flat, table_f32)
    labels = lax.bitcast_convert_type(labels_f32, jnp.bfloat16).reshape(
        NUM_TOKENS, HIDDEN
    )
    x = generated_hidden_states.reshape(NUM_TOKENS, GEN_HIDDEN)
    out = _tc_mse(x, W, b.reshape(1, HIDDEN), labels)
    return out.reshape(()).astype(jnp.bfloat16)


# speedup vs baseline: 2.3938x; 2.3938x over previous
import jax
import jax.numpy as jnp
from jax import lax
from jax.experimental import pallas as pl
from jax.experimental.pallas import tpu as pltpu
from jax.experimental.pallas import tpu_sc as plsc

VOCAB = 8192
HIDDEN = 256
GEN_HIDDEN = 4096
NUM_TOKENS = 8192
HALF = HIDDEN // 2

NUM_WORKERS = 32
TOK_PER_W = NUM_TOKENS // NUM_WORKERS
IDX_CHUNK = 128

TILE_M = 1024
GRID_M = NUM_TOKENS // TILE_M


def _sc_gather_body(ids_hbm, table_hbm, out_hbm, idx_v, rows_v, sem):
    wid = lax.axis_index("s") * 2 + lax.axis_index("c")
    base = wid * TOK_PER_W
    pltpu.sync_copy(ids_hbm.at[pl.ds(base, TOK_PER_W)], idx_v)
    for i in range(TOK_PER_W // 16):
        sl = pl.ds(i * 16, 16)
        idx_v[sl] = idx_v[sl] - 4
    copies = []
    for j in range(TOK_PER_W // IDX_CHUNK):
        sl = pl.ds(j * IDX_CHUNK, IDX_CHUNK)
        copies.append(
            pltpu.async_copy(table_hbm.at[idx_v.at[sl]], rows_v.at[sl], sem)
        )
    for c in copies:
        c.wait()
    pltpu.sync_copy(rows_v, out_hbm.at[pl.ds(base, TOK_PER_W)])


def _sc_gather(ids_flat, table_i32):
    mesh = plsc.VectorSubcoreMesh(core_axis_name="c", subcore_axis_name="s")
    fn = pl.kernel(
        _sc_gather_body,
        out_type=jax.ShapeDtypeStruct((NUM_TOKENS, HALF), jnp.int32),
        mesh=mesh,
        scratch_types=[
            pltpu.VMEM((TOK_PER_W,), jnp.int32),
            pltpu.VMEM((TOK_PER_W, HALF), jnp.int32),
            pltpu.SemaphoreType.DMA,
        ],
    )
    return fn(ids_flat, table_i32)


def _tc_mse_body(x_ref, w_ref, b_ref, l_ref, out_ref):
    i = pl.program_id(0)
    f32 = jnp.float32
    bf16 = jnp.bfloat16
    acc = lax.dot_general(
        x_ref[...], w_ref[...],
        dimension_numbers=(((1,), (1,)), ((), ())),
        preferred_element_type=f32,
    ).astype(bf16)
    feat = (acc.astype(f32) + b_ref[...].astype(f32)).astype(bf16)
    words = l_ref[...]
    lab_lo = pltpu.bitcast(words << 16, f32)
    lab_hi = pltpu.bitcast(words & jnp.int32(-65536), f32)
    f_lo = feat[:, :HALF].astype(f32)
    f_hi = feat[:, HALF:].astype(f32)
    d_lo = (f_lo - lab_lo).astype(bf16).astype(f32)
    d_hi = (f_hi - lab_hi).astype(bf16).astype(f32)
    part = jnp.sum((d_lo * d_lo).astype(bf16).astype(f32)) + jnp.sum(
        (d_hi * d_hi).astype(bf16).astype(f32)
    )

    @pl.when(i == 0)
    def _():
        out_ref[0, 0] = 0.0

    out_ref[0, 0] += part

    @pl.when(i == GRID_M - 1)
    def _():
        out_ref[0, 0] = out_ref[0, 0] * (1.0 / (NUM_TOKENS * HIDDEN))


def _tc_mse(x, w, b2d, labels_i32):
    return pl.pallas_call(
        _tc_mse_body,
        grid=(GRID_M,),
        in_specs=[
            pl.BlockSpec((TILE_M, GEN_HIDDEN), lambda i: (i, 0)),
            pl.BlockSpec((HIDDEN, GEN_HIDDEN), lambda i: (0, 0)),
            pl.BlockSpec((1, HIDDEN), lambda i: (0, 0)),
            pl.BlockSpec((TILE_M, HALF), lambda i: (i, 0)),
        ],
        out_specs=pl.BlockSpec(memory_space=pltpu.SMEM),
        out_shape=jax.ShapeDtypeStruct((1, 1), jnp.float32),
    )(x, w, b2d, labels_i32)


def kernel(input_ids, generated_hidden_states, embedding, W, b):
    ids_flat = input_ids.reshape(NUM_TOKENS).astype(jnp.int32)
    emb_u16 = lax.bitcast_convert_type(embedding, jnp.uint16)
    lo = emb_u16[:, :HALF].astype(jnp.uint32)
    hi = emb_u16[:, HALF:].astype(jnp.uint32)
    table_i32 = lax.bitcast_convert_type(lo | (hi << 16), jnp.int32)
    labels_i32 = _sc_gather(ids_flat, table_i32)
    x = generated_hidden_states.reshape(NUM_TOKENS, GEN_HIDDEN)
    out = _tc_mse(x, W, b.reshape(1, HIDDEN), labels_i32)
    return out.reshape(()).astype(jnp.bfloat16)
